# baseline (speedup 1.0000x reference)
import dataclasses
import functools

import jax
import jax.numpy as jnp
from jax import lax
from jax.experimental import pallas as pl
from jax.experimental.pallas import tpu as pltpu
from jax.experimental.pallas import tpu_sc as plsc

N_NODES = 10000
N_EDGES = 320000
NC = 2
NS = 16
LANES = 16
ROWS_PER_TILE = N_NODES // NS
EDGES_PER_TILE = N_EDGES // (NC * NS)
CHUNK = 80
NCHUNKS = EDGES_PER_TILE // CHUNK
DEG_CHUNK = 16
DEG_NCHUNKS = EDGES_PER_TILE // DEG_CHUNK

_VEC_MESH = plsc.VectorSubcoreMesh(core_axis_name="c", subcore_axis_name="s")

_SC_PARAMS = pltpu.CompilerParams()
if "needs_layout_passes" in pltpu.CompilerParams.__dataclass_fields__:
    _SC_PARAMS = dataclasses.replace(_SC_PARAMS, needs_layout_passes=False)


def _splat(vec16, j):
    lane = lax.iota(jnp.int32, LANES)
    scl = jnp.sum(jnp.where(lane == j, vec16, 0.0))
    return jnp.full((LANES,), scl, dtype=jnp.float32)


DEG_NB = 4


def _sc_deg(edge_attr, col):
    @functools.partial(
        pl.kernel,
        out_type=jax.ShapeDtypeStruct((NC, NS, ROWS_PER_TILE, 128), jnp.float32),
        mesh=_VEC_MESH,
        compiler_params=_SC_PARAMS,
        scratch_types=[
            pltpu.VMEM_SHARED((N_NODES, 128), jnp.float32),
            pltpu.VMEM((DEG_NB * CHUNK,), jnp.float32),
            pltpu.VMEM((DEG_NB * CHUNK, 128), jnp.float32),
            pltpu.VMEM((DEG_NB, CHUNK), jnp.int32),
            pltpu.VMEM((25, 128), jnp.float32),
            pltpu.SemaphoreType.DMA((DEG_NB,)),
            pltpu.SemaphoreType.DMA((DEG_NB,)),
        ],
    )
    def deg_kernel(attr_hbm, col_hbm, deg_hbm,
                   acc_sh, av, awide_v, cidx_v, zb, sem_i, sem_s):
        c = lax.axis_index("c")
        s = lax.axis_index("s")
        t = c * NS + s
        ept = t * EDGES_PER_TILE

        def awide_b(b):
            return awide_v.at[pl.ds(b * CHUNK, CHUNK)]

        def idx_copies(k, b):
            base = ept + k * CHUNK
            return (
                pltpu.make_async_copy(
                    attr_hbm.at[pl.ds(base, CHUNK)],
                    av.at[pl.ds(b * CHUNK, CHUNK)],
                    sem_i.at[b],
                ),
                pltpu.make_async_copy(
                    col_hbm.at[pl.ds(base, CHUNK)], cidx_v.at[b], sem_i.at[b]
                ),
            )

        def scatter_copy(b):
            return pltpu.make_async_copy(
                awide_b(b), acc_sh.at[cidx_v.at[b]], sem_s.at[b]
            )

        def build(b):
            @plsc.parallel_loop(0, CHUNK // LANES, unroll=2)
            def _g(gi):
                a16 = av[pl.ds(b * CHUNK + gi * LANES, LANES)]
                for j in range(LANES):
                    w = _splat(a16, j)
                    r = b * CHUNK + gi * LANES + j
                    for q in range(8):
                        awide_v[r, pl.ds(q * LANES, LANES)] = w

        @pl.loop(0, 25)
        def _zero(r):
            for q in range(8):
                zb[r, pl.ds(q * LANES, LANES)] = jnp.zeros((LANES,), jnp.float32)

        for j in range(ROWS_PER_TILE // 25):
            pltpu.sync_copy(zb, acc_sh.at[pl.ds(s * ROWS_PER_TILE + j * 25, 25)])
        plsc.subcore_barrier()

        for cp in idx_copies(0, 0):
            cp.start()
        for cp in idx_copies(1, 1):
            cp.start()
        for cp in idx_copies(0, 0):
            cp.wait()

        @pl.loop(0, NCHUNKS + 3, step=DEG_NB)
        def _slot(i):
            for b in range(DEG_NB):
                k = i + b
                b1 = (b + 1) % DEG_NB
                b2 = (b + 2) % DEG_NB

                @pl.when(k + 1 < NCHUNKS)
                def _():
                    for cp in idx_copies(k + 1, b1):
                        cp.wait()

                @pl.when(k >= DEG_NB - 1)
                def _():
                    scatter_copy(b1).wait()

                @pl.when(k + 2 < NCHUNKS)
                def _():
                    for cp in idx_copies(k + 2, b2):
                        cp.start()

                @pl.when(k < NCHUNKS)
                def _():
                    build(b)
                    scatter_copy(b).start(add=True)

        plsc.subcore_barrier()
        pltpu.sync_copy(
            acc_sh.at[pl.ds(s * ROWS_PER_TILE, ROWS_PER_TILE)], deg_hbm.at[c, s]
        )

    return deg_kernel(edge_attr, col)


AGG_NB = 4


def _sc_aggregate(z, row, col, edge_attr):
    @functools.partial(
        pl.kernel,
        out_type=jax.ShapeDtypeStruct((NC, NS, ROWS_PER_TILE, 128), jnp.float32),
        mesh=_VEC_MESH,
        compiler_params=_SC_PARAMS,
        scratch_types=[
            pltpu.VMEM_SHARED((N_NODES, 128), jnp.float32),
            pltpu.VMEM((AGG_NB * CHUNK, 128), jnp.float32),
            pltpu.VMEM((AGG_NB * CHUNK,), jnp.float32),
            pltpu.VMEM((AGG_NB, CHUNK), jnp.int32),
            pltpu.VMEM((AGG_NB, CHUNK), jnp.int32),
            pltpu.VMEM((25, 128), jnp.float32),
            pltpu.SemaphoreType.DMA((AGG_NB,)),
            pltpu.SemaphoreType.DMA((AGG_NB,)),
            pltpu.SemaphoreType.DMA((AGG_NB,)),
        ],
    )
    def agg_kernel(z_hbm, row_hbm, col_hbm, attr_hbm, part_hbm,
                   acc_sh, rows_v, av, ridx_v, cidx_v, zb,
                   sem_i, sem_g, sem_s):
        c = lax.axis_index("c")
        s = lax.axis_index("s")
        t = c * NS + s
        ept = t * EDGES_PER_TILE

        def rows_b(b):
            return rows_v.at[pl.ds(b * CHUNK, CHUNK)]

        def av_b(b):
            return av.at[pl.ds(b * CHUNK, CHUNK)]

        def idx_copies(k, b):
            base = ept + k * CHUNK
            return (
                pltpu.make_async_copy(
                    row_hbm.at[pl.ds(base, CHUNK)], ridx_v.at[b], sem_i.at[b]
                ),
                pltpu.make_async_copy(
                    col_hbm.at[pl.ds(base, CHUNK)], cidx_v.at[b], sem_i.at[b]
                ),
                pltpu.make_async_copy(
                    attr_hbm.at[pl.ds(base, CHUNK)], av_b(b), sem_i.at[b]
                ),
            )

        def issue_idx(k, b):
            for cp in idx_copies(k, b):
                cp.start()

        def wait_idx(k, b):
            for cp in idx_copies(k, b):
                cp.wait()

        def gather_copy(b):
            return pltpu.make_async_copy(
                z_hbm.at[ridx_v.at[b]], rows_b(b), sem_g.at[b]
            )

        def scatter_copy(b):
            return pltpu.make_async_copy(
                rows_b(b), acc_sh.at[cidx_v.at[b]], sem_s.at[b]
            )

        def scale(b):
            @plsc.parallel_loop(0, CHUNK // LANES, unroll=2)
            def _g(gi):
                a16 = av[pl.ds(b * CHUNK + gi * LANES, LANES)]
                for j in range(LANES):
                    w = _splat(a16, j)
                    r = b * CHUNK + gi * LANES + j
                    for q in range(8):
                        rows_v[r, pl.ds(q * LANES, LANES)] = (
                            rows_v[r, pl.ds(q * LANES, LANES)] * w
                        )

        @pl.loop(0, 25)
        def _zero(r):
            for q in range(8):
                zb[r, pl.ds(q * LANES, LANES)] = jnp.zeros((LANES,), jnp.float32)

        for j in range(ROWS_PER_TILE // 25):
            pltpu.sync_copy(zb, acc_sh.at[pl.ds(s * ROWS_PER_TILE + j * 25, 25)])
        plsc.subcore_barrier()

        issue_idx(0, 0)
        issue_idx(1, 1)
        wait_idx(0, 0)
        gather_copy(0).start()

        @pl.loop(0, NCHUNKS + 3, step=AGG_NB)
        def _slot(i):
            for b in range(AGG_NB):
                k = i + b
                b1 = (b + 1) % AGG_NB
                b2 = (b + 2) % AGG_NB

                @pl.when(k + 1 < NCHUNKS)
                def _():
                    wait_idx(k + 1, b1)

                @pl.when(k >= 3)
                def _():
                    scatter_copy(b1).wait()

                @pl.when(k + 1 < NCHUNKS)
                def _():
                    gather_copy(b1).start()

                @pl.when(k + 2 < NCHUNKS)
                def _():
                    issue_idx(k + 2, b2)

                @pl.when(k < NCHUNKS)
                def _():
                    gather_copy(b).wait()
                    scale(b)
                    scatter_copy(b).start(add=True)

        plsc.subcore_barrier()
        pltpu.sync_copy(
            acc_sh.at[pl.ds(s * ROWS_PER_TILE, ROWS_PER_TILE)], part_hbm.at[c, s]
        )

    return agg_kernel(z, row, col, edge_attr)


def _tc_inv_max(edge_attr):
    def body(a_ref, o_ref):
        o_ref[...] = 1.0 / jnp.max(a_ref[...]) * jnp.ones((1, 1), jnp.float32)

    return pl.pallas_call(
        body,
        out_shape=jax.ShapeDtypeStruct((1, 1), jnp.float32),
    )(edge_attr.reshape(2500, 128))


_BM = 2000


def _tc_dis(p0, p1, x, inv):
    def body(inv_ref, p0_ref, p1_ref, x_ref, g_ref, dis2_ref, z0_ref):
        inv_m = inv_ref[0, 0]
        deg = (p0_ref[:, 0:1] + p1_ref[:, 0:1]) * inv_m + 2.0
        dis = lax.rsqrt(deg)
        g = dis * jnp.sqrt(inv_m)
        g_ref[...] = g
        dis2_ref[...] = 2.0 * dis * dis
        z0_ref[...] = g * x_ref[...]

    grid = (N_NODES // _BM,)
    return pl.pallas_call(
        body,
        grid=grid,
        in_specs=[
            pl.BlockSpec(memory_space=pltpu.SMEM),
            pl.BlockSpec((_BM, 128), lambda i: (i, 0)),
            pl.BlockSpec((_BM, 128), lambda i: (i, 0)),
            pl.BlockSpec((_BM, 128), lambda i: (i, 0)),
        ],
        out_specs=[
            pl.BlockSpec((_BM, 1), lambda i: (i, 0)),
            pl.BlockSpec((_BM, 1), lambda i: (i, 0)),
            pl.BlockSpec((_BM, 128), lambda i: (i, 0)),
        ],
        out_shape=[
            jax.ShapeDtypeStruct((N_NODES, 1), jnp.float32),
            jax.ShapeDtypeStruct((N_NODES, 1), jnp.float32),
            jax.ShapeDtypeStruct((N_NODES, 128), jnp.float32),
        ],
    )(inv, p0, p1, x)


def _tc_layer0(a0, a1, x, g, dis2, W0, b0, W1):
    def body(a0_ref, a1_ref, x_ref, g_ref, dis2_ref, W0_ref, b0_ref, W1_ref,
             v1_ref, z1a_ref, z1b_ref):
        u0 = g_ref[...] * (a0_ref[...] + a1_ref[...]) + dis2_ref[...] * x_ref[...]
        h1 = jnp.maximum(
            jnp.dot(u0, W0_ref[...], preferred_element_type=jnp.float32)
            + b0_ref[...],
            0.0,
        )
        v1 = jnp.dot(h1, W1_ref[...], preferred_element_type=jnp.float32)
        v1_ref[...] = v1
        z1 = g_ref[...] * v1
        z1a_ref[...] = z1[:, :128]
        z1b_ref[...] = z1[:, 128:]

    grid = (N_NODES // _BM,)
    return pl.pallas_call(
        body,
        grid=grid,
        in_specs=[
            pl.BlockSpec((_BM, 128), lambda i: (i, 0)),
            pl.BlockSpec((_BM, 128), lambda i: (i, 0)),
            pl.BlockSpec((_BM, 128), lambda i: (i, 0)),
            pl.BlockSpec((_BM, 1), lambda i: (i, 0)),
            pl.BlockSpec((_BM, 1), lambda i: (i, 0)),
            pl.BlockSpec((128, 256), lambda i: (0, 0)),
            pl.BlockSpec((1, 256), lambda i: (0, 0)),
            pl.BlockSpec((256, 256), lambda i: (0, 0)),
        ],
        out_specs=[
            pl.BlockSpec((_BM, 256), lambda i: (i, 0)),
            pl.BlockSpec((_BM, 128), lambda i: (i, 0)),
            pl.BlockSpec((_BM, 128), lambda i: (i, 0)),
        ],
        out_shape=[
            jax.ShapeDtypeStruct((N_NODES, 256), jnp.float32),
            jax.ShapeDtypeStruct((N_NODES, 128), jnp.float32),
            jax.ShapeDtypeStruct((N_NODES, 128), jnp.float32),
        ],
    )(a0, a1, x, g, dis2, W0, b0, W1)


def _tc_layer1(aa0, aa1, ab0, ab1, v1, g, dis2, b1, W2):
    def body(aa0_ref, aa1_ref, ab0_ref, ab1_ref, v1_ref, g_ref, dis2_ref,
             b1_ref, W2_ref, v2_ref, z2_ref):
        g_v = g_ref[...]
        d2 = dis2_ref[...]
        ua = g_v * (aa0_ref[...] + aa1_ref[...]) + d2 * v1_ref[:, :128]
        ub = g_v * (ab0_ref[...] + ab1_ref[...]) + d2 * v1_ref[:, 128:]
        h2 = jnp.maximum(
            jnp.concatenate([ua, ub], axis=1) + b1_ref[...], 0.0
        )
        v2 = jnp.dot(h2, W2_ref[...], preferred_element_type=jnp.float32)
        v2_ref[...] = v2
        z2_ref[...] = g_v * v2

    grid = (N_NODES // _BM,)
    return pl.pallas_call(
        body,
        grid=grid,
        in_specs=[
            pl.BlockSpec((_BM, 128), lambda i: (i, 0)),
            pl.BlockSpec((_BM, 128), lambda i: (i, 0)),
            pl.BlockSpec((_BM, 128), lambda i: (i, 0)),
            pl.BlockSpec((_BM, 128), lambda i: (i, 0)),
            pl.BlockSpec((_BM, 256), lambda i: (i, 0)),
            pl.BlockSpec((_BM, 1), lambda i: (i, 0)),
            pl.BlockSpec((_BM, 1), lambda i: (i, 0)),
            pl.BlockSpec((1, 256), lambda i: (0, 0)),
            pl.BlockSpec((256, 128), lambda i: (0, 0)),
        ],
        out_specs=[
            pl.BlockSpec((_BM, 128), lambda i: (i, 0)),
            pl.BlockSpec((_BM, 128), lambda i: (i, 0)),
        ],
        out_shape=[
            jax.ShapeDtypeStruct((N_NODES, 128), jnp.float32),
            jax.ShapeDtypeStruct((N_NODES, 128), jnp.float32),
        ],
    )(aa0, aa1, ab0, ab1, v1, g, dis2, b1, W2)


def _tc_layer2(a0, a1, v2, g, dis2, b2):
    def body(a0_ref, a1_ref, v2_ref, g_ref, dis2_ref, b2_ref, o_ref):
        o_ref[...] = (
            g_ref[...] * (a0_ref[...] + a1_ref[...])
            + dis2_ref[...] * v2_ref[...]
            + b2_ref[...]
        )

    grid = (N_NODES // _BM,)
    return pl.pallas_call(
        body,
        grid=grid,
        in_specs=[
            pl.BlockSpec((_BM, 128), lambda i: (i, 0)),
            pl.BlockSpec((_BM, 128), lambda i: (i, 0)),
            pl.BlockSpec((_BM, 128), lambda i: (i, 0)),
            pl.BlockSpec((_BM, 1), lambda i: (i, 0)),
            pl.BlockSpec((_BM, 1), lambda i: (i, 0)),
            pl.BlockSpec((1, 128), lambda i: (0, 0)),
        ],
        out_specs=pl.BlockSpec((_BM, 128), lambda i: (i, 0)),
        out_shape=jax.ShapeDtypeStruct((N_NODES, 128), jnp.float32),
    )(a0, a1, v2, g, dis2, b2)


def kernel(x, edge_index, edge_attr, W0, b0, W1, b1, W2, b2):
    row = edge_index[0].astype(jnp.int32)
    col = edge_index[1].astype(jnp.int32)

    inv = _tc_inv_max(edge_attr)
    degp = _sc_deg(edge_attr, col)
    degp = degp.reshape(NC, N_NODES, 128)
    g, dis2, z0 = _tc_dis(degp[0], degp[1], x, inv)

    acc0 = _sc_aggregate(z0, row, col, edge_attr).reshape(NC, N_NODES, 128)
    v1, z1a, z1b = _tc_layer0(acc0[0], acc0[1], x, g, dis2, W0,
                              b0.reshape(1, 256), W1)

    acc1a = _sc_aggregate(z1a, row, col, edge_attr).reshape(NC, N_NODES, 128)
    acc1b = _sc_aggregate(z1b, row, col, edge_attr).reshape(NC, N_NODES, 128)
    v2, z2 = _tc_layer1(acc1a[0], acc1a[1], acc1b[0], acc1b[1], v1, g, dis2,
                        b1.reshape(1, 256), W2)

    acc2 = _sc_aggregate(z2, row, col, edge_attr).reshape(NC, N_NODES, 128)
    return _tc_layer2(acc2[0], acc2[1], v2, g, dis2, b2.reshape(1, 128))


# speedup vs baseline: 1.1567x; 1.1567x over previous
import dataclasses
import functools

import jax
import jax.numpy as jnp
from jax import lax
from jax.experimental import pallas as pl
from jax.experimental.pallas import tpu as pltpu
from jax.experimental.pallas import tpu_sc as plsc

N_NODES = 10000
N_EDGES = 320000
NC = 2
NS = 16
LANES = 16
ROWS_PER_TILE = N_NODES // NS
EDGES_PER_TILE = N_EDGES // (NC * NS)
CHUNK = 80
NCHUNKS = EDGES_PER_TILE // CHUNK
DEG_CHUNK = 16
DEG_NCHUNKS = EDGES_PER_TILE // DEG_CHUNK

_VEC_MESH = plsc.VectorSubcoreMesh(core_axis_name="c", subcore_axis_name="s")

_SC_PARAMS = pltpu.CompilerParams()
if "needs_layout_passes" in pltpu.CompilerParams.__dataclass_fields__:
    _SC_PARAMS = dataclasses.replace(_SC_PARAMS, needs_layout_passes=False)


_GATHER_DNUMS = lax.GatherDimensionNumbers(
    offset_dims=(), collapsed_slice_dims=(0,), start_index_map=(0,)
)


def _splat(vec16, j):
    idx = jnp.full((LANES, 1), j, dtype=jnp.int32)
    return lax.gather(vec16, idx, _GATHER_DNUMS, (1,),
                      mode=lax.GatherScatterMode.PROMISE_IN_BOUNDS)


DEG_NB = 4


def _sc_deg(edge_attr, col):
    @functools.partial(
        pl.kernel,
        out_type=jax.ShapeDtypeStruct((NC, NS, ROWS_PER_TILE, 128), jnp.float32),
        mesh=_VEC_MESH,
        compiler_params=_SC_PARAMS,
        scratch_types=[
            pltpu.VMEM_SHARED((N_NODES, 128), jnp.float32),
            pltpu.VMEM((DEG_NB * CHUNK,), jnp.float32),
            pltpu.VMEM((DEG_NB * CHUNK, 128), jnp.float32),
            pltpu.VMEM((DEG_NB, CHUNK), jnp.int32),
            pltpu.VMEM((25, 128), jnp.float32),
            pltpu.SemaphoreType.DMA((DEG_NB,)),
            pltpu.SemaphoreType.DMA((DEG_NB,)),
        ],
    )
    def deg_kernel(attr_hbm, col_hbm, deg_hbm,
                   acc_sh, av, awide_v, cidx_v, zb, sem_i, sem_s):
        c = lax.axis_index("c")
        s = lax.axis_index("s")
        t = c * NS + s
        ept = t * EDGES_PER_TILE

        def awide_b(b):
            return awide_v.at[pl.ds(b * CHUNK, CHUNK)]

        def idx_copies(k, b):
            base = ept + k * CHUNK
            return (
                pltpu.make_async_copy(
                    attr_hbm.at[pl.ds(base, CHUNK)],
                    av.at[pl.ds(b * CHUNK, CHUNK)],
                    sem_i.at[b],
                ),
                pltpu.make_async_copy(
                    col_hbm.at[pl.ds(base, CHUNK)], cidx_v.at[b], sem_i.at[b]
                ),
            )

        def scatter_copy(b):
            return pltpu.make_async_copy(
                awide_b(b), acc_sh.at[cidx_v.at[b]], sem_s.at[b]
            )

        def build(b):
            @pl.loop(0, CHUNK // LANES)
            def _g(gi):
                a16 = av[pl.ds(b * CHUNK + gi * LANES, LANES)]
                for j in range(LANES):
                    w = _splat(a16, j)
                    r = b * CHUNK + gi * LANES + j
                    for q in range(8):
                        awide_v[r, pl.ds(q * LANES, LANES)] = w

        @pl.loop(0, 25)
        def _zero(r):
            for q in range(8):
                zb[r, pl.ds(q * LANES, LANES)] = jnp.zeros((LANES,), jnp.float32)

        for j in range(ROWS_PER_TILE // 25):
            pltpu.sync_copy(zb, acc_sh.at[pl.ds(s * ROWS_PER_TILE + j * 25, 25)])
        plsc.subcore_barrier()

        for cp in idx_copies(0, 0):
            cp.start()
        for cp in idx_copies(1, 1):
            cp.start()
        for cp in idx_copies(0, 0):
            cp.wait()

        @pl.loop(0, NCHUNKS + 3, step=DEG_NB)
        def _slot(i):
            for b in range(DEG_NB):
                k = i + b
                b1 = (b + 1) % DEG_NB
                b2 = (b + 2) % DEG_NB

                @pl.when(k + 1 < NCHUNKS)
                def _():
                    for cp in idx_copies(k + 1, b1):
                        cp.wait()

                @pl.when(k >= DEG_NB - 1)
                def _():
                    scatter_copy(b1).wait()

                @pl.when(k + 2 < NCHUNKS)
                def _():
                    for cp in idx_copies(k + 2, b2):
                        cp.start()

                @pl.when(k < NCHUNKS)
                def _():
                    build(b)
                    scatter_copy(b).start(add=True)

        plsc.subcore_barrier()
        pltpu.sync_copy(
            acc_sh.at[pl.ds(s * ROWS_PER_TILE, ROWS_PER_TILE)], deg_hbm.at[c, s]
        )

    return deg_kernel(edge_attr, col)


AGG_NB = 4


def _sc_aggregate(z, row, col, edge_attr):
    @functools.partial(
        pl.kernel,
        out_type=jax.ShapeDtypeStruct((NC, NS, ROWS_PER_TILE, 128), jnp.float32),
        mesh=_VEC_MESH,
        compiler_params=_SC_PARAMS,
        scratch_types=[
            pltpu.VMEM_SHARED((N_NODES, 128), jnp.float32),
            pltpu.VMEM((AGG_NB * CHUNK, 128), jnp.float32),
            pltpu.VMEM((AGG_NB * CHUNK,), jnp.float32),
            pltpu.VMEM((AGG_NB, CHUNK), jnp.int32),
            pltpu.VMEM((AGG_NB, CHUNK), jnp.int32),
            pltpu.VMEM((25, 128), jnp.float32),
            pltpu.SemaphoreType.DMA((AGG_NB,)),
            pltpu.SemaphoreType.DMA((AGG_NB,)),
            pltpu.SemaphoreType.DMA((AGG_NB,)),
        ],
    )
    def agg_kernel(z_hbm, row_hbm, col_hbm, attr_hbm, part_hbm,
                   acc_sh, rows_v, av, ridx_v, cidx_v, zb,
                   sem_i, sem_g, sem_s):
        c = lax.axis_index("c")
        s = lax.axis_index("s")
        t = c * NS + s
        ept = t * EDGES_PER_TILE

        def rows_b(b):
            return rows_v.at[pl.ds(b * CHUNK, CHUNK)]

        def av_b(b):
            return av.at[pl.ds(b * CHUNK, CHUNK)]

        def idx_copies(k, b):
            base = ept + k * CHUNK
            return (
                pltpu.make_async_copy(
                    row_hbm.at[pl.ds(base, CHUNK)], ridx_v.at[b], sem_i.at[b]
                ),
                pltpu.make_async_copy(
                    col_hbm.at[pl.ds(base, CHUNK)], cidx_v.at[b], sem_i.at[b]
                ),
                pltpu.make_async_copy(
                    attr_hbm.at[pl.ds(base, CHUNK)], av_b(b), sem_i.at[b]
                ),
            )

        def issue_idx(k, b):
            for cp in idx_copies(k, b):
                cp.start()

        def wait_idx(k, b):
            for cp in idx_copies(k, b):
                cp.wait()

        def gather_copy(b):
            return pltpu.make_async_copy(
                z_hbm.at[ridx_v.at[b]], rows_b(b), sem_g.at[b]
            )

        def scatter_copy(b):
            return pltpu.make_async_copy(
                rows_b(b), acc_sh.at[cidx_v.at[b]], sem_s.at[b]
            )

        def scale(b):
            @pl.loop(0, CHUNK // LANES)
            def _g(gi):
                a16 = av[pl.ds(b * CHUNK + gi * LANES, LANES)]
                for j in range(LANES):
                    w = _splat(a16, j)
                    r = b * CHUNK + gi * LANES + j
                    for q in range(8):
                        rows_v[r, pl.ds(q * LANES, LANES)] = (
                            rows_v[r, pl.ds(q * LANES, LANES)] * w
                        )

        @pl.loop(0, 25)
        def _zero(r):
            for q in range(8):
                zb[r, pl.ds(q * LANES, LANES)] = jnp.zeros((LANES,), jnp.float32)

        for j in range(ROWS_PER_TILE // 25):
            pltpu.sync_copy(zb, acc_sh.at[pl.ds(s * ROWS_PER_TILE + j * 25, 25)])
        plsc.subcore_barrier()

        issue_idx(0, 0)
        issue_idx(1, 1)
        wait_idx(0, 0)
        gather_copy(0).start()

        @pl.loop(0, NCHUNKS + 3, step=AGG_NB)
        def _slot(i):
            for b in range(AGG_NB):
                k = i + b
                b1 = (b + 1) % AGG_NB
                b2 = (b + 2) % AGG_NB

                @pl.when(k + 1 < NCHUNKS)
                def _():
                    wait_idx(k + 1, b1)

                @pl.when(k >= 3)
                def _():
                    scatter_copy(b1).wait()

                @pl.when(k + 1 < NCHUNKS)
                def _():
                    gather_copy(b1).start()

                @pl.when(k + 2 < NCHUNKS)
                def _():
                    issue_idx(k + 2, b2)

                @pl.when(k < NCHUNKS)
                def _():
                    gather_copy(b).wait()
                    scale(b)
                    scatter_copy(b).start(add=True)

        plsc.subcore_barrier()
        pltpu.sync_copy(
            acc_sh.at[pl.ds(s * ROWS_PER_TILE, ROWS_PER_TILE)], part_hbm.at[c, s]
        )

    return agg_kernel(z, row, col, edge_attr)


def _tc_inv_max(edge_attr):
    def body(a_ref, o_ref):
        o_ref[...] = 1.0 / jnp.max(a_ref[...]) * jnp.ones((1, 1), jnp.float32)

    return pl.pallas_call(
        body,
        out_shape=jax.ShapeDtypeStruct((1, 1), jnp.float32),
    )(edge_attr.reshape(2500, 128))


_BM = 2000


def _tc_dis(p0, p1, x, inv):
    def body(inv_ref, p0_ref, p1_ref, x_ref, g_ref, dis2_ref, z0_ref):
        inv_m = inv_ref[0, 0]
        deg = (p0_ref[:, 0:1] + p1_ref[:, 0:1]) * inv_m + 2.0
        dis = lax.rsqrt(deg)
        g = dis * jnp.sqrt(inv_m)
        g_ref[...] = g
        dis2_ref[...] = 2.0 * dis * dis
        z0_ref[...] = g * x_ref[...]

    grid = (N_NODES // _BM,)
    return pl.pallas_call(
        body,
        grid=grid,
        in_specs=[
            pl.BlockSpec(memory_space=pltpu.SMEM),
            pl.BlockSpec((_BM, 128), lambda i: (i, 0)),
            pl.BlockSpec((_BM, 128), lambda i: (i, 0)),
            pl.BlockSpec((_BM, 128), lambda i: (i, 0)),
        ],
        out_specs=[
            pl.BlockSpec((_BM, 1), lambda i: (i, 0)),
            pl.BlockSpec((_BM, 1), lambda i: (i, 0)),
            pl.BlockSpec((_BM, 128), lambda i: (i, 0)),
        ],
        out_shape=[
            jax.ShapeDtypeStruct((N_NODES, 1), jnp.float32),
            jax.ShapeDtypeStruct((N_NODES, 1), jnp.float32),
            jax.ShapeDtypeStruct((N_NODES, 128), jnp.float32),
        ],
    )(inv, p0, p1, x)


def _tc_layer0(a0, a1, x, g, dis2, W0, b0, W1):
    def body(a0_ref, a1_ref, x_ref, g_ref, dis2_ref, W0_ref, b0_ref, W1_ref,
             v1_ref, z1a_ref, z1b_ref):
        u0 = g_ref[...] * (a0_ref[...] + a1_ref[...]) + dis2_ref[...] * x_ref[...]
        h1 = jnp.maximum(
            jnp.dot(u0, W0_ref[...], preferred_element_type=jnp.float32)
            + b0_ref[...],
            0.0,
        )
        v1 = jnp.dot(h1, W1_ref[...], preferred_element_type=jnp.float32)
        v1_ref[...] = v1
        z1 = g_ref[...] * v1
        z1a_ref[...] = z1[:, :128]
        z1b_ref[...] = z1[:, 128:]

    grid = (N_NODES // _BM,)
    return pl.pallas_call(
        body,
        grid=grid,
        in_specs=[
            pl.BlockSpec((_BM, 128), lambda i: (i, 0)),
            pl.BlockSpec((_BM, 128), lambda i: (i, 0)),
            pl.BlockSpec((_BM, 128), lambda i: (i, 0)),
            pl.BlockSpec((_BM, 1), lambda i: (i, 0)),
            pl.BlockSpec((_BM, 1), lambda i: (i, 0)),
            pl.BlockSpec((128, 256), lambda i: (0, 0)),
            pl.BlockSpec((1, 256), lambda i: (0, 0)),
            pl.BlockSpec((256, 256), lambda i: (0, 0)),
        ],
        out_specs=[
            pl.BlockSpec((_BM, 256), lambda i: (i, 0)),
            pl.BlockSpec((_BM, 128), lambda i: (i, 0)),
            pl.BlockSpec((_BM, 128), lambda i: (i, 0)),
        ],
        out_shape=[
            jax.ShapeDtypeStruct((N_NODES, 256), jnp.float32),
            jax.ShapeDtypeStruct((N_NODES, 128), jnp.float32),
            jax.ShapeDtypeStruct((N_NODES, 128), jnp.float32),
        ],
    )(a0, a1, x, g, dis2, W0, b0, W1)


def _tc_layer1(aa0, aa1, ab0, ab1, v1, g, dis2, b1, W2):
    def body(aa0_ref, aa1_ref, ab0_ref, ab1_ref, v1_ref, g_ref, dis2_ref,
             b1_ref, W2_ref, v2_ref, z2_ref):
        g_v = g_ref[...]
        d2 = dis2_ref[...]
        ua = g_v * (aa0_ref[...] + aa1_ref[...]) + d2 * v1_ref[:, :128]
        ub = g_v * (ab0_ref[...] + ab1_ref[...]) + d2 * v1_ref[:, 128:]
        h2 = jnp.maximum(
            jnp.concatenate([ua, ub], axis=1) + b1_ref[...], 0.0
        )
        v2 = jnp.dot(h2, W2_ref[...], preferred_element_type=jnp.float32)
        v2_ref[...] = v2
        z2_ref[...] = g_v * v2

    grid = (N_NODES // _BM,)
    return pl.pallas_call(
        body,
        grid=grid,
        in_specs=[
            pl.BlockSpec((_BM, 128), lambda i: (i, 0)),
            pl.BlockSpec((_BM, 128), lambda i: (i, 0)),
            pl.BlockSpec((_BM, 128), lambda i: (i, 0)),
            pl.BlockSpec((_BM, 128), lambda i: (i, 0)),
            pl.BlockSpec((_BM, 256), lambda i: (i, 0)),
            pl.BlockSpec((_BM, 1), lambda i: (i, 0)),
            pl.BlockSpec((_BM, 1), lambda i: (i, 0)),
            pl.BlockSpec((1, 256), lambda i: (0, 0)),
            pl.BlockSpec((256, 128), lambda i: (0, 0)),
        ],
        out_specs=[
            pl.BlockSpec((_BM, 128), lambda i: (i, 0)),
            pl.BlockSpec((_BM, 128), lambda i: (i, 0)),
        ],
        out_shape=[
            jax.ShapeDtypeStruct((N_NODES, 128), jnp.float32),
            jax.ShapeDtypeStruct((N_NODES, 128), jnp.float32),
        ],
    )(aa0, aa1, ab0, ab1, v1, g, dis2, b1, W2)


def _tc_layer2(a0, a1, v2, g, dis2, b2):
    def body(a0_ref, a1_ref, v2_ref, g_ref, dis2_ref, b2_ref, o_ref):
        o_ref[...] = (
            g_ref[...] * (a0_ref[...] + a1_ref[...])
            + dis2_ref[...] * v2_ref[...]
            + b2_ref[...]
        )

    grid = (N_NODES // _BM,)
    return pl.pallas_call(
        body,
        grid=grid,
        in_specs=[
            pl.BlockSpec((_BM, 128), lambda i: (i, 0)),
            pl.BlockSpec((_BM, 128), lambda i: (i, 0)),
            pl.BlockSpec((_BM, 128), lambda i: (i, 0)),
            pl.BlockSpec((_BM, 1), lambda i: (i, 0)),
            pl.BlockSpec((_BM, 1), lambda i: (i, 0)),
            pl.BlockSpec((1, 128), lambda i: (0, 0)),
        ],
        out_specs=pl.BlockSpec((_BM, 128), lambda i: (i, 0)),
        out_shape=jax.ShapeDtypeStruct((N_NODES, 128), jnp.float32),
    )(a0, a1, v2, g, dis2, b2)


def kernel(x, edge_index, edge_attr, W0, b0, W1, b1, W2, b2):
    row = edge_index[0].astype(jnp.int32)
    col = edge_index[1].astype(jnp.int32)

    inv = _tc_inv_max(edge_attr)
    degp = _sc_deg(edge_attr, col)
    degp = degp.reshape(NC, N_NODES, 128)
    g, dis2, z0 = _tc_dis(degp[0], degp[1], x, inv)

    acc0 = _sc_aggregate(z0, row, col, edge_attr).reshape(NC, N_NODES, 128)
    v1, z1a, z1b = _tc_layer0(acc0[0], acc0[1], x, g, dis2, W0,
                              b0.reshape(1, 256), W1)

    acc1a = _sc_aggregate(z1a, row, col, edge_attr).reshape(NC, N_NODES, 128)
    acc1b = _sc_aggregate(z1b, row, col, edge_attr).reshape(NC, N_NODES, 128)
    v2, z2 = _tc_layer1(acc1a[0], acc1a[1], acc1b[0], acc1b[1], v1, g, dis2,
                        b1.reshape(1, 256), W2)

    acc2 = _sc_aggregate(z2, row, col, edge_attr).reshape(NC, N_NODES, 128)
    return _tc_layer2(acc2[0], acc2[1], v2, g, dis2, b2.reshape(1, 128))


# speedup vs baseline: 1.2112x; 1.0471x over previous
import dataclasses
import functools

import jax
import jax.numpy as jnp
from jax import lax
from jax.experimental import pallas as pl
from jax.experimental.pallas import tpu as pltpu
from jax.experimental.pallas import tpu_sc as plsc

N_NODES = 10000
N_EDGES = 320000
NC = 2
NS = 16
LANES = 16
ROWS_PER_TILE = N_NODES // NS
EDGES_PER_TILE = N_EDGES // (NC * NS)
CHUNK = 80
NCHUNKS = EDGES_PER_TILE // CHUNK
DEG_CHUNK = 16
DEG_NCHUNKS = EDGES_PER_TILE // DEG_CHUNK

_VEC_MESH = plsc.VectorSubcoreMesh(core_axis_name="c", subcore_axis_name="s")

_SC_PARAMS = pltpu.CompilerParams()
if "needs_layout_passes" in pltpu.CompilerParams.__dataclass_fields__:
    _SC_PARAMS = dataclasses.replace(_SC_PARAMS, needs_layout_passes=False)


_GATHER_DNUMS = lax.GatherDimensionNumbers(
    offset_dims=(), collapsed_slice_dims=(0,), start_index_map=(0,)
)


def _splat(vec16, j):
    idx = jnp.full((LANES, 1), j, dtype=jnp.int32)
    return lax.gather(vec16, idx, _GATHER_DNUMS, (1,),
                      mode=lax.GatherScatterMode.PROMISE_IN_BOUNDS)


DEG_NB = 4


def _sc_deg(edge_attr, col):
    @functools.partial(
        pl.kernel,
        out_type=jax.ShapeDtypeStruct((NC, NS, ROWS_PER_TILE, 128), jnp.float32),
        mesh=_VEC_MESH,
        compiler_params=_SC_PARAMS,
        scratch_types=[
            pltpu.VMEM_SHARED((N_NODES, 128), jnp.float32),
            pltpu.VMEM((DEG_NB * CHUNK,), jnp.float32),
            pltpu.VMEM((DEG_NB * CHUNK, 128), jnp.float32),
            pltpu.VMEM((DEG_NB, CHUNK), jnp.int32),
            pltpu.VMEM((25, 128), jnp.float32),
            pltpu.SemaphoreType.DMA((DEG_NB,)),
            pltpu.SemaphoreType.DMA((DEG_NB,)),
        ],
    )
    def deg_kernel(attr_hbm, col_hbm, deg_hbm,
                   acc_sh, av, awide_v, cidx_v, zb, sem_i, sem_s):
        c = lax.axis_index("c")
        s = lax.axis_index("s")
        t = c * NS + s
        ept = t * EDGES_PER_TILE

        def awide_b(b):
            return awide_v.at[pl.ds(b * CHUNK, CHUNK)]

        def idx_copies(k, b):
            base = ept + k * CHUNK
            return (
                pltpu.make_async_copy(
                    attr_hbm.at[pl.ds(base, CHUNK)],
                    av.at[pl.ds(b * CHUNK, CHUNK)],
                    sem_i.at[b],
                ),
                pltpu.make_async_copy(
                    col_hbm.at[pl.ds(base, CHUNK)], cidx_v.at[b], sem_i.at[b]
                ),
            )

        def scatter_copy(b):
            return pltpu.make_async_copy(
                awide_b(b), acc_sh.at[cidx_v.at[b]], sem_s.at[b]
            )

        def build(b):
            @pl.loop(0, CHUNK // LANES)
            def _g(gi):
                a16 = av[pl.ds(b * CHUNK + gi * LANES, LANES)]
                for j in range(LANES):
                    w = _splat(a16, j)
                    r = b * CHUNK + gi * LANES + j
                    for q in range(8):
                        awide_v[r, pl.ds(q * LANES, LANES)] = w

        @pl.loop(0, 25)
        def _zero(r):
            for q in range(8):
                zb[r, pl.ds(q * LANES, LANES)] = jnp.zeros((LANES,), jnp.float32)

        for j in range(ROWS_PER_TILE // 25):
            pltpu.sync_copy(zb, acc_sh.at[pl.ds(s * ROWS_PER_TILE + j * 25, 25)])
        plsc.subcore_barrier()

        for cp in idx_copies(0, 0):
            cp.start()
        for cp in idx_copies(1, 1):
            cp.start()
        for cp in idx_copies(0, 0):
            cp.wait()

        @pl.loop(0, NCHUNKS + 3, step=DEG_NB)
        def _slot(i):
            for b in range(DEG_NB):
                k = i + b
                b1 = (b + 1) % DEG_NB
                b2 = (b + 2) % DEG_NB

                @pl.when(k + 1 < NCHUNKS)
                def _():
                    for cp in idx_copies(k + 1, b1):
                        cp.wait()

                @pl.when(k >= DEG_NB - 1)
                def _():
                    scatter_copy(b1).wait()

                @pl.when(k + 2 < NCHUNKS)
                def _():
                    for cp in idx_copies(k + 2, b2):
                        cp.start()

                @pl.when(k < NCHUNKS)
                def _():
                    build(b)
                    scatter_copy(b).start(add=True)

        plsc.subcore_barrier()
        pltpu.sync_copy(
            acc_sh.at[pl.ds(s * ROWS_PER_TILE, ROWS_PER_TILE)], deg_hbm.at[c, s]
        )

    return deg_kernel(edge_attr, col)


AGG_NB = 4


def _sc_aggregate(z, row, col, edge_attr):
    @functools.partial(
        pl.kernel,
        out_type=jax.ShapeDtypeStruct((NC, NS, ROWS_PER_TILE, 128), jnp.float32),
        mesh=_VEC_MESH,
        compiler_params=_SC_PARAMS,
        scratch_types=[
            pltpu.VMEM_SHARED((N_NODES, 128), jnp.float32),
            pltpu.VMEM((AGG_NB * CHUNK, 128), jnp.float32),
            pltpu.VMEM((AGG_NB * CHUNK,), jnp.float32),
            pltpu.VMEM((AGG_NB, CHUNK), jnp.int32),
            pltpu.VMEM((AGG_NB, CHUNK), jnp.int32),
            pltpu.VMEM((25, 128), jnp.float32),
            pltpu.SemaphoreType.DMA((AGG_NB,)),
            pltpu.SemaphoreType.DMA((AGG_NB,)),
            pltpu.SemaphoreType.DMA((AGG_NB,)),
        ],
    )
    def agg_kernel(z_hbm, row_hbm, col_hbm, attr_hbm, part_hbm,
                   acc_sh, rows_v, av, ridx_v, cidx_v, zb,
                   sem_i, sem_g, sem_s):
        c = lax.axis_index("c")
        s = lax.axis_index("s")
        t = c * NS + s
        ept = t * EDGES_PER_TILE

        def rows_b(b):
            return rows_v.at[pl.ds(b * CHUNK, CHUNK)]

        def av_b(b):
            return av.at[pl.ds(b * CHUNK, CHUNK)]

        def idx_copies(k, b):
            base = ept + k * CHUNK
            return (
                pltpu.make_async_copy(
                    row_hbm.at[pl.ds(base, CHUNK)], ridx_v.at[b], sem_i.at[b]
                ),
                pltpu.make_async_copy(
                    col_hbm.at[pl.ds(base, CHUNK)], cidx_v.at[b], sem_i.at[b]
                ),
                pltpu.make_async_copy(
                    attr_hbm.at[pl.ds(base, CHUNK)], av_b(b), sem_i.at[b]
                ),
            )

        def issue_idx(k, b):
            for cp in idx_copies(k, b):
                cp.start()

        def wait_idx(k, b):
            for cp in idx_copies(k, b):
                cp.wait()

        def gather_copy(b):
            return pltpu.make_async_copy(
                z_hbm.at[ridx_v.at[b]], rows_b(b), sem_g.at[b]
            )

        def scatter_copy(b):
            return pltpu.make_async_copy(
                rows_b(b), acc_sh.at[cidx_v.at[b]], sem_s.at[b]
            )

        def scale(b):
            @pl.loop(0, CHUNK // LANES)
            def _g(gi):
                a16 = av[pl.ds(b * CHUNK + gi * LANES, LANES)]
                for j in range(LANES):
                    w = _splat(a16, j)
                    r = b * CHUNK + gi * LANES + j
                    for q in range(8):
                        rows_v[r, pl.ds(q * LANES, LANES)] = (
                            rows_v[r, pl.ds(q * LANES, LANES)] * w
                        )

        @pl.loop(0, 25)
        def _zero(r):
            for q in range(8):
                zb[r, pl.ds(q * LANES, LANES)] = jnp.zeros((LANES,), jnp.float32)

        for j in range(ROWS_PER_TILE // 25):
            pltpu.sync_copy(zb, acc_sh.at[pl.ds(s * ROWS_PER_TILE + j * 25, 25)])
        plsc.subcore_barrier()

        issue_idx(0, 0)
        issue_idx(1, 1)
        wait_idx(0, 0)
        gather_copy(0).start()

        @pl.loop(0, NCHUNKS + 3, step=AGG_NB)
        def _slot(i):
            for b in range(AGG_NB):
                k = i + b
                b1 = (b + 1) % AGG_NB
                b2 = (b + 2) % AGG_NB

                @pl.when(k + 1 < NCHUNKS)
                def _():
                    wait_idx(k + 1, b1)

                @pl.when(k >= 3)
                def _():
                    scatter_copy(b1).wait()

                @pl.when(k + 1 < NCHUNKS)
                def _():
                    gather_copy(b1).start()

                @pl.when(k + 2 < NCHUNKS)
                def _():
                    issue_idx(k + 2, b2)

                @pl.when(k < NCHUNKS)
                def _():
                    gather_copy(b).wait()
                    scale(b)
                    scatter_copy(b).start(add=True)

        plsc.subcore_barrier()
        pltpu.sync_copy(
            acc_sh.at[pl.ds(s * ROWS_PER_TILE, ROWS_PER_TILE)], part_hbm.at[c, s]
        )

    return agg_kernel(z, row, col, edge_attr)


EPT_PAIR = N_EDGES // NS
NCH_PAIR = EPT_PAIR // CHUNK


def _sc_aggregate_pair(z_pair, row, col, edge_attr):
    @functools.partial(
        pl.kernel,
        out_type=jax.ShapeDtypeStruct((NC, NS, ROWS_PER_TILE, 128), jnp.float32),
        mesh=_VEC_MESH,
        compiler_params=_SC_PARAMS,
        scratch_types=[
            pltpu.VMEM_SHARED((N_NODES, 128), jnp.float32),
            pltpu.VMEM((AGG_NB * CHUNK, 128), jnp.float32),
            pltpu.VMEM((AGG_NB * CHUNK,), jnp.float32),
            pltpu.VMEM((AGG_NB, CHUNK), jnp.int32),
            pltpu.VMEM((AGG_NB, CHUNK), jnp.int32),
            pltpu.VMEM((25, 128), jnp.float32),
            pltpu.SemaphoreType.DMA((AGG_NB,)),
            pltpu.SemaphoreType.DMA((AGG_NB,)),
            pltpu.SemaphoreType.DMA((AGG_NB,)),
        ],
    )
    def agg2_kernel(z_hbm, row_hbm, col_hbm, attr_hbm, part_hbm,
                    acc_sh, rows_v, av, ridx_v, cidx_v, zb,
                    sem_i, sem_g, sem_s):
        c = lax.axis_index("c")
        s = lax.axis_index("s")
        zc = z_hbm.at[c]
        ept = s * EPT_PAIR

        def rows_b(b):
            return rows_v.at[pl.ds(b * CHUNK, CHUNK)]

        def av_b(b):
            return av.at[pl.ds(b * CHUNK, CHUNK)]

        def idx_copies(k, b):
            base = ept + k * CHUNK
            return (
                pltpu.make_async_copy(
                    row_hbm.at[pl.ds(base, CHUNK)], ridx_v.at[b], sem_i.at[b]
                ),
                pltpu.make_async_copy(
                    col_hbm.at[pl.ds(base, CHUNK)], cidx_v.at[b], sem_i.at[b]
                ),
                pltpu.make_async_copy(
                    attr_hbm.at[pl.ds(base, CHUNK)], av_b(b), sem_i.at[b]
                ),
            )

        def issue_idx(k, b):
            for cp in idx_copies(k, b):
                cp.start()

        def wait_idx(k, b):
            for cp in idx_copies(k, b):
                cp.wait()

        def gather_copy(b):
            return pltpu.make_async_copy(
                zc.at[ridx_v.at[b]], rows_b(b), sem_g.at[b]
            )

        def scatter_copy(b):
            return pltpu.make_async_copy(
                rows_b(b), acc_sh.at[cidx_v.at[b]], sem_s.at[b]
            )

        def scale(b):
            @pl.loop(0, CHUNK // LANES)
            def _g(gi):
                a16 = av[pl.ds(b * CHUNK + gi * LANES, LANES)]
                for j in range(LANES):
                    w = _splat(a16, j)
                    r = b * CHUNK + gi * LANES + j
                    for q in range(8):
                        rows_v[r, pl.ds(q * LANES, LANES)] = (
                            rows_v[r, pl.ds(q * LANES, LANES)] * w
                        )

        @pl.loop(0, 25)
        def _zero(r):
            for q in range(8):
                zb[r, pl.ds(q * LANES, LANES)] = jnp.zeros((LANES,), jnp.float32)

        for j in range(ROWS_PER_TILE // 25):
            pltpu.sync_copy(zb, acc_sh.at[pl.ds(s * ROWS_PER_TILE + j * 25, 25)])
        plsc.subcore_barrier()

        issue_idx(0, 0)
        issue_idx(1, 1)
        wait_idx(0, 0)
        gather_copy(0).start()

        @pl.loop(0, NCH_PAIR + 3, step=AGG_NB)
        def _slot(i):
            for b in range(AGG_NB):
                k = i + b
                b1 = (b + 1) % AGG_NB
                b2 = (b + 2) % AGG_NB

                @pl.when(k + 1 < NCH_PAIR)
                def _():
                    wait_idx(k + 1, b1)

                @pl.when(jnp.logical_and(k >= 3, k < NCH_PAIR + 3))
                def _():
                    scatter_copy(b1).wait()

                @pl.when(k + 1 < NCH_PAIR)
                def _():
                    gather_copy(b1).start()

                @pl.when(k + 2 < NCH_PAIR)
                def _():
                    issue_idx(k + 2, b2)

                @pl.when(k < NCH_PAIR)
                def _():
                    gather_copy(b).wait()
                    scale(b)
                    scatter_copy(b).start(add=True)

        plsc.subcore_barrier()
        pltpu.sync_copy(
            acc_sh.at[pl.ds(s * ROWS_PER_TILE, ROWS_PER_TILE)], part_hbm.at[c, s]
        )

    return agg2_kernel(z_pair, row, col, edge_attr)


_BM = 2000


def _tc_dis(p0, p1, x, edge_attr):
    def body(attr_ref, p0_ref, p1_ref, x_ref, g_ref, dis2_ref, z0_ref, inv_ref):
        @pl.when(pl.program_id(0) == 0)
        def _():
            inv_ref[0] = 1.0 / jnp.max(attr_ref[...])

        inv_m = inv_ref[0]
        deg = (p0_ref[:, 0:1] + p1_ref[:, 0:1]) * inv_m + 2.0
        dis = lax.rsqrt(deg)
        g = dis * jnp.sqrt(inv_m)
        g_ref[...] = g
        dis2_ref[...] = 2.0 * dis * dis
        z0_ref[...] = g * x_ref[...]

    grid = (N_NODES // _BM,)
    return pl.pallas_call(
        body,
        grid=grid,
        in_specs=[
            pl.BlockSpec((2500, 128), lambda i: (0, 0)),
            pl.BlockSpec((_BM, 128), lambda i: (i, 0)),
            pl.BlockSpec((_BM, 128), lambda i: (i, 0)),
            pl.BlockSpec((_BM, 128), lambda i: (i, 0)),
        ],
        out_specs=[
            pl.BlockSpec((_BM, 1), lambda i: (i, 0)),
            pl.BlockSpec((_BM, 1), lambda i: (i, 0)),
            pl.BlockSpec((_BM, 128), lambda i: (i, 0)),
        ],
        out_shape=[
            jax.ShapeDtypeStruct((N_NODES, 1), jnp.float32),
            jax.ShapeDtypeStruct((N_NODES, 1), jnp.float32),
            jax.ShapeDtypeStruct((N_NODES, 128), jnp.float32),
        ],
        scratch_shapes=[pltpu.SMEM((1,), jnp.float32)],
    )(edge_attr.reshape(2500, 128), p0, p1, x)


def _tc_layer0(a0, a1, x, g, dis2, W0, b0, W1):
    def body(a0_ref, a1_ref, x_ref, g_ref, dis2_ref, W0_ref, b0_ref, W1_ref,
             v1_ref, z1_ref):
        u0 = g_ref[...] * (a0_ref[...] + a1_ref[...]) + dis2_ref[...] * x_ref[...]
        h1 = jnp.maximum(
            jnp.dot(u0, W0_ref[...], preferred_element_type=jnp.float32)
            + b0_ref[...],
            0.0,
        )
        v1 = jnp.dot(h1, W1_ref[...], preferred_element_type=jnp.float32)
        v1_ref[...] = v1
        z1 = g_ref[...] * v1
        z1_ref[0] = z1[:, :128]
        z1_ref[1] = z1[:, 128:]

    grid = (N_NODES // _BM,)
    return pl.pallas_call(
        body,
        grid=grid,
        in_specs=[
            pl.BlockSpec((_BM, 128), lambda i: (i, 0)),
            pl.BlockSpec((_BM, 128), lambda i: (i, 0)),
            pl.BlockSpec((_BM, 128), lambda i: (i, 0)),
            pl.BlockSpec((_BM, 1), lambda i: (i, 0)),
            pl.BlockSpec((_BM, 1), lambda i: (i, 0)),
            pl.BlockSpec((128, 256), lambda i: (0, 0)),
            pl.BlockSpec((1, 256), lambda i: (0, 0)),
            pl.BlockSpec((256, 256), lambda i: (0, 0)),
        ],
        out_specs=[
            pl.BlockSpec((_BM, 256), lambda i: (i, 0)),
            pl.BlockSpec((2, _BM, 128), lambda i: (0, i, 0)),
        ],
        out_shape=[
            jax.ShapeDtypeStruct((N_NODES, 256), jnp.float32),
            jax.ShapeDtypeStruct((2, N_NODES, 128), jnp.float32),
        ],
    )(a0, a1, x, g, dis2, W0, b0, W1)


def _tc_layer1(aa, ab, v1, g, dis2, b1, W2):
    def body(aa_ref, ab_ref, v1_ref, g_ref, dis2_ref, b1_ref, W2_ref,
             v2_ref, z2_ref):
        g_v = g_ref[...]
        d2 = dis2_ref[...]
        ua = g_v * aa_ref[...] + d2 * v1_ref[:, :128]
        ub = g_v * ab_ref[...] + d2 * v1_ref[:, 128:]
        h2 = jnp.maximum(
            jnp.concatenate([ua, ub], axis=1) + b1_ref[...], 0.0
        )
        v2 = jnp.dot(h2, W2_ref[...], preferred_element_type=jnp.float32)
        v2_ref[...] = v2
        z2_ref[...] = g_v * v2

    grid = (N_NODES // _BM,)
    return pl.pallas_call(
        body,
        grid=grid,
        in_specs=[
            pl.BlockSpec((_BM, 128), lambda i: (i, 0)),
            pl.BlockSpec((_BM, 128), lambda i: (i, 0)),
            pl.BlockSpec((_BM, 256), lambda i: (i, 0)),
            pl.BlockSpec((_BM, 1), lambda i: (i, 0)),
            pl.BlockSpec((_BM, 1), lambda i: (i, 0)),
            pl.BlockSpec((1, 256), lambda i: (0, 0)),
            pl.BlockSpec((256, 128), lambda i: (0, 0)),
        ],
        out_specs=[
            pl.BlockSpec((_BM, 128), lambda i: (i, 0)),
            pl.BlockSpec((_BM, 128), lambda i: (i, 0)),
        ],
        out_shape=[
            jax.ShapeDtypeStruct((N_NODES, 128), jnp.float32),
            jax.ShapeDtypeStruct((N_NODES, 128), jnp.float32),
        ],
    )(aa, ab, v1, g, dis2, b1, W2)


def _tc_layer2(a0, a1, v2, g, dis2, b2):
    def body(a0_ref, a1_ref, v2_ref, g_ref, dis2_ref, b2_ref, o_ref):
        o_ref[...] = (
            g_ref[...] * (a0_ref[...] + a1_ref[...])
            + dis2_ref[...] * v2_ref[...]
            + b2_ref[...]
        )

    grid = (N_NODES // _BM,)
    return pl.pallas_call(
        body,
        grid=grid,
        in_specs=[
            pl.BlockSpec((_BM, 128), lambda i: (i, 0)),
            pl.BlockSpec((_BM, 128), lambda i: (i, 0)),
            pl.BlockSpec((_BM, 128), lambda i: (i, 0)),
            pl.BlockSpec((_BM, 1), lambda i: (i, 0)),
            pl.BlockSpec((_BM, 1), lambda i: (i, 0)),
            pl.BlockSpec((1, 128), lambda i: (0, 0)),
        ],
        out_specs=pl.BlockSpec((_BM, 128), lambda i: (i, 0)),
        out_shape=jax.ShapeDtypeStruct((N_NODES, 128), jnp.float32),
    )(a0, a1, v2, g, dis2, b2)


def kernel(x, edge_index, edge_attr, W0, b0, W1, b1, W2, b2):
    row = edge_index[0].astype(jnp.int32)
    col = edge_index[1].astype(jnp.int32)

    degp = _sc_deg(edge_attr, col)
    degp = degp.reshape(NC, N_NODES, 128)
    g, dis2, z0 = _tc_dis(degp[0], degp[1], x, edge_attr)

    acc0 = _sc_aggregate(z0, row, col, edge_attr).reshape(NC, N_NODES, 128)
    v1, z1 = _tc_layer0(acc0[0], acc0[1], x, g, dis2, W0,
                        b0.reshape(1, 256), W1)

    acc1 = _sc_aggregate_pair(z1, row, col, edge_attr).reshape(NC, N_NODES, 128)
    v2, z2 = _tc_layer1(acc1[0], acc1[1], v1, g, dis2,
                        b1.reshape(1, 256), W2)

    acc2 = _sc_aggregate(z2, row, col, edge_attr).reshape(NC, N_NODES, 128)
    return _tc_layer2(acc2[0], acc2[1], v2, g, dis2, b2.reshape(1, 128))


# speedup vs baseline: 1.2166x; 1.0045x over previous
import dataclasses
import functools

import jax
import jax.numpy as jnp
from jax import lax
from jax.experimental import pallas as pl
from jax.experimental.pallas import tpu as pltpu
from jax.experimental.pallas import tpu_sc as plsc

N_NODES = 10000
N_EDGES = 320000
NC = 2
NS = 16
LANES = 16
ROWS_PER_TILE = N_NODES // NS
EDGES_PER_TILE = N_EDGES // (NC * NS)
CHUNK = 80
NCHUNKS = EDGES_PER_TILE // CHUNK
DEG_CHUNK = 16
DEG_NCHUNKS = EDGES_PER_TILE // DEG_CHUNK

_VEC_MESH = plsc.VectorSubcoreMesh(core_axis_name="c", subcore_axis_name="s")

_SC_PARAMS = pltpu.CompilerParams()
if "needs_layout_passes" in pltpu.CompilerParams.__dataclass_fields__:
    _SC_PARAMS = dataclasses.replace(_SC_PARAMS, needs_layout_passes=False)


_GATHER_DNUMS = lax.GatherDimensionNumbers(
    offset_dims=(), collapsed_slice_dims=(0,), start_index_map=(0,)
)


def _splat(vec16, j):
    idx = jnp.full((LANES, 1), j, dtype=jnp.int32)
    return lax.gather(vec16, idx, _GATHER_DNUMS, (1,),
                      mode=lax.GatherScatterMode.PROMISE_IN_BOUNDS)


DEG_NB = 4


def _sc_deg(edge_attr, col):
    @functools.partial(
        pl.kernel,
        out_type=jax.ShapeDtypeStruct((NC, NS, ROWS_PER_TILE, 128), jnp.float32),
        mesh=_VEC_MESH,
        compiler_params=_SC_PARAMS,
        scratch_types=[
            pltpu.VMEM_SHARED((N_NODES, 128), jnp.float32),
            pltpu.VMEM((DEG_NB * CHUNK,), jnp.float32),
            pltpu.VMEM((DEG_NB * CHUNK, 128), jnp.float32),
            pltpu.VMEM((DEG_NB, CHUNK), jnp.int32),
            pltpu.VMEM((25, 128), jnp.float32),
            pltpu.SemaphoreType.DMA((DEG_NB,)),
            pltpu.SemaphoreType.DMA((DEG_NB,)),
        ],
    )
    def deg_kernel(attr_hbm, col_hbm, deg_hbm,
                   acc_sh, av, awide_v, cidx_v, zb, sem_i, sem_s):
        c = lax.axis_index("c")
        s = lax.axis_index("s")
        t = c * NS + s
        ept = t * EDGES_PER_TILE

        def awide_b(b):
            return awide_v.at[pl.ds(b * CHUNK, CHUNK)]

        def idx_copies(k, b):
            base = ept + k * CHUNK
            return (
                pltpu.make_async_copy(
                    attr_hbm.at[pl.ds(base, CHUNK)],
                    av.at[pl.ds(b * CHUNK, CHUNK)],
                    sem_i.at[b],
                ),
                pltpu.make_async_copy(
                    col_hbm.at[pl.ds(base, CHUNK)], cidx_v.at[b], sem_i.at[b]
                ),
            )

        def scatter_copy(b):
            return pltpu.make_async_copy(
                awide_b(b), acc_sh.at[cidx_v.at[b]], sem_s.at[b]
            )

        def build(b):
            @pl.loop(0, CHUNK // LANES)
            def _g(gi):
                a16 = av[pl.ds(b * CHUNK + gi * LANES, LANES)]
                for j in range(LANES):
                    w = _splat(a16, j)
                    r = b * CHUNK + gi * LANES + j
                    for q in range(8):
                        awide_v[r, pl.ds(q * LANES, LANES)] = w

        @pl.loop(0, 25)
        def _zero(r):
            for q in range(8):
                zb[r, pl.ds(q * LANES, LANES)] = jnp.zeros((LANES,), jnp.float32)

        zero_cps = [
            pltpu.make_async_copy(
                zb,
                acc_sh.at[pl.ds(s * ROWS_PER_TILE + j * 25, 25)],
                sem_i.at[0],
            )
            for j in range(ROWS_PER_TILE // 25)
        ]
        for cp in zero_cps:
            cp.start()
        for cp in zero_cps:
            cp.wait()
        plsc.subcore_barrier()

        for cp in idx_copies(0, 0):
            cp.start()
        for cp in idx_copies(1, 1):
            cp.start()
        for cp in idx_copies(0, 0):
            cp.wait()

        @pl.loop(0, NCHUNKS + 3, step=DEG_NB)
        def _slot(i):
            for b in range(DEG_NB):
                k = i + b
                b1 = (b + 1) % DEG_NB
                b2 = (b + 2) % DEG_NB

                @pl.when(k + 1 < NCHUNKS)
                def _():
                    for cp in idx_copies(k + 1, b1):
                        cp.wait()

                @pl.when(k >= DEG_NB - 1)
                def _():
                    scatter_copy(b1).wait()

                @pl.when(k + 2 < NCHUNKS)
                def _():
                    for cp in idx_copies(k + 2, b2):
                        cp.start()

                @pl.when(k < NCHUNKS)
                def _():
                    build(b)
                    scatter_copy(b).start(add=True)

        plsc.subcore_barrier()
        pltpu.sync_copy(
            acc_sh.at[pl.ds(s * ROWS_PER_TILE, ROWS_PER_TILE)], deg_hbm.at[c, s]
        )

    return deg_kernel(edge_attr, col)


AGG_NB = 4


def _sc_aggregate(z, row, col, edge_attr):
    @functools.partial(
        pl.kernel,
        out_type=jax.ShapeDtypeStruct((NC, NS, ROWS_PER_TILE, 128), jnp.float32),
        mesh=_VEC_MESH,
        compiler_params=_SC_PARAMS,
        scratch_types=[
            pltpu.VMEM_SHARED((N_NODES, 128), jnp.float32),
            pltpu.VMEM((AGG_NB * CHUNK, 128), jnp.float32),
            pltpu.VMEM((AGG_NB * CHUNK,), jnp.float32),
            pltpu.VMEM((AGG_NB, CHUNK), jnp.int32),
            pltpu.VMEM((AGG_NB, CHUNK), jnp.int32),
            pltpu.VMEM((25, 128), jnp.float32),
            pltpu.SemaphoreType.DMA((AGG_NB,)),
            pltpu.SemaphoreType.DMA((AGG_NB,)),
            pltpu.SemaphoreType.DMA((AGG_NB,)),
        ],
    )
    def agg_kernel(z_hbm, row_hbm, col_hbm, attr_hbm, part_hbm,
                   acc_sh, rows_v, av, ridx_v, cidx_v, zb,
                   sem_i, sem_g, sem_s):
        c = lax.axis_index("c")
        s = lax.axis_index("s")
        t = c * NS + s
        ept = t * EDGES_PER_TILE

        def rows_b(b):
            return rows_v.at[pl.ds(b * CHUNK, CHUNK)]

        def av_b(b):
            return av.at[pl.ds(b * CHUNK, CHUNK)]

        def idx_copies(k, b):
            base = ept + k * CHUNK
            return (
                pltpu.make_async_copy(
                    row_hbm.at[pl.ds(base, CHUNK)], ridx_v.at[b], sem_i.at[b]
                ),
                pltpu.make_async_copy(
                    col_hbm.at[pl.ds(base, CHUNK)], cidx_v.at[b], sem_i.at[b]
                ),
                pltpu.make_async_copy(
                    attr_hbm.at[pl.ds(base, CHUNK)], av_b(b), sem_i.at[b]
                ),
            )

        def issue_idx(k, b):
            for cp in idx_copies(k, b):
                cp.start()

        def wait_idx(k, b):
            for cp in idx_copies(k, b):
                cp.wait()

        def gather_copy(b):
            return pltpu.make_async_copy(
                z_hbm.at[ridx_v.at[b]], rows_b(b), sem_g.at[b]
            )

        def scatter_copy(b):
            return pltpu.make_async_copy(
                rows_b(b), acc_sh.at[cidx_v.at[b]], sem_s.at[b]
            )

        def scale(b):
            @pl.loop(0, CHUNK // LANES)
            def _g(gi):
                a16 = av[pl.ds(b * CHUNK + gi * LANES, LANES)]
                for j in range(LANES):
                    w = _splat(a16, j)
                    r = b * CHUNK + gi * LANES + j
                    for q in range(8):
                        rows_v[r, pl.ds(q * LANES, LANES)] = (
                            rows_v[r, pl.ds(q * LANES, LANES)] * w
                        )

        @pl.loop(0, 25)
        def _zero(r):
            for q in range(8):
                zb[r, pl.ds(q * LANES, LANES)] = jnp.zeros((LANES,), jnp.float32)

        zero_cps = [
            pltpu.make_async_copy(
                zb,
                acc_sh.at[pl.ds(s * ROWS_PER_TILE + j * 25, 25)],
                sem_i.at[0],
            )
            for j in range(ROWS_PER_TILE // 25)
        ]
        for cp in zero_cps:
            cp.start()
        for cp in zero_cps:
            cp.wait()
        plsc.subcore_barrier()

        issue_idx(0, 0)
        issue_idx(1, 1)
        wait_idx(0, 0)
        gather_copy(0).start()

        @pl.loop(0, NCHUNKS + 3, step=AGG_NB)
        def _slot(i):
            for b in range(AGG_NB):
                k = i + b
                b1 = (b + 1) % AGG_NB
                b2 = (b + 2) % AGG_NB

                @pl.when(k + 1 < NCHUNKS)
                def _():
                    wait_idx(k + 1, b1)

                @pl.when(k >= 3)
                def _():
                    scatter_copy(b1).wait()

                @pl.when(k + 1 < NCHUNKS)
                def _():
                    gather_copy(b1).start()

                @pl.when(k + 2 < NCHUNKS)
                def _():
                    issue_idx(k + 2, b2)

                @pl.when(k < NCHUNKS)
                def _():
                    gather_copy(b).wait()
                    scale(b)
                    scatter_copy(b).start(add=True)

        plsc.subcore_barrier()
        pltpu.sync_copy(
            acc_sh.at[pl.ds(s * ROWS_PER_TILE, ROWS_PER_TILE)], part_hbm.at[c, s]
        )

    return agg_kernel(z, row, col, edge_attr)


EPT_PAIR = N_EDGES // NS
NCH_PAIR = EPT_PAIR // CHUNK


def _sc_aggregate_pair(z_pair, row, col, edge_attr):
    @functools.partial(
        pl.kernel,
        out_type=jax.ShapeDtypeStruct((NC, NS, ROWS_PER_TILE, 128), jnp.float32),
        mesh=_VEC_MESH,
        compiler_params=_SC_PARAMS,
        scratch_types=[
            pltpu.VMEM_SHARED((N_NODES, 128), jnp.float32),
            pltpu.VMEM((AGG_NB * CHUNK, 128), jnp.float32),
            pltpu.VMEM((AGG_NB * CHUNK,), jnp.float32),
            pltpu.VMEM((AGG_NB, CHUNK), jnp.int32),
            pltpu.VMEM((AGG_NB, CHUNK), jnp.int32),
            pltpu.VMEM((25, 128), jnp.float32),
            pltpu.SemaphoreType.DMA((AGG_NB,)),
            pltpu.SemaphoreType.DMA((AGG_NB,)),
            pltpu.SemaphoreType.DMA((AGG_NB,)),
        ],
    )
    def agg2_kernel(z_hbm, row_hbm, col_hbm, attr_hbm, part_hbm,
                    acc_sh, rows_v, av, ridx_v, cidx_v, zb,
                    sem_i, sem_g, sem_s):
        c = lax.axis_index("c")
        s = lax.axis_index("s")
        zc = z_hbm.at[c]
        ept = s * EPT_PAIR

        def rows_b(b):
            return rows_v.at[pl.ds(b * CHUNK, CHUNK)]

        def av_b(b):
            return av.at[pl.ds(b * CHUNK, CHUNK)]

        def idx_copies(k, b):
            base = ept + k * CHUNK
            return (
                pltpu.make_async_copy(
                    row_hbm.at[pl.ds(base, CHUNK)], ridx_v.at[b], sem_i.at[b]
                ),
                pltpu.make_async_copy(
                    col_hbm.at[pl.ds(base, CHUNK)], cidx_v.at[b], sem_i.at[b]
                ),
                pltpu.make_async_copy(
                    attr_hbm.at[pl.ds(base, CHUNK)], av_b(b), sem_i.at[b]
                ),
            )

        def issue_idx(k, b):
            for cp in idx_copies(k, b):
                cp.start()

        def wait_idx(k, b):
            for cp in idx_copies(k, b):
                cp.wait()

        def gather_copy(b):
            return pltpu.make_async_copy(
                zc.at[ridx_v.at[b]], rows_b(b), sem_g.at[b]
            )

        def scatter_copy(b):
            return pltpu.make_async_copy(
                rows_b(b), acc_sh.at[cidx_v.at[b]], sem_s.at[b]
            )

        def scale(b):
            @pl.loop(0, CHUNK // LANES)
            def _g(gi):
                a16 = av[pl.ds(b * CHUNK + gi * LANES, LANES)]
                for j in range(LANES):
                    w = _splat(a16, j)
                    r = b * CHUNK + gi * LANES + j
                    for q in range(8):
                        rows_v[r, pl.ds(q * LANES, LANES)] = (
                            rows_v[r, pl.ds(q * LANES, LANES)] * w
                        )

        @pl.loop(0, 25)
        def _zero(r):
            for q in range(8):
                zb[r, pl.ds(q * LANES, LANES)] = jnp.zeros((LANES,), jnp.float32)

        zero_cps = [
            pltpu.make_async_copy(
                zb,
                acc_sh.at[pl.ds(s * ROWS_PER_TILE + j * 25, 25)],
                sem_i.at[0],
            )
            for j in range(ROWS_PER_TILE // 25)
        ]
        for cp in zero_cps:
            cp.start()
        for cp in zero_cps:
            cp.wait()
        plsc.subcore_barrier()

        issue_idx(0, 0)
        issue_idx(1, 1)
        wait_idx(0, 0)
        gather_copy(0).start()

        @pl.loop(0, NCH_PAIR + 3, step=AGG_NB)
        def _slot(i):
            for b in range(AGG_NB):
                k = i + b
                b1 = (b + 1) % AGG_NB
                b2 = (b + 2) % AGG_NB

                @pl.when(k + 1 < NCH_PAIR)
                def _():
                    wait_idx(k + 1, b1)

                @pl.when(jnp.logical_and(k >= 3, k < NCH_PAIR + 3))
                def _():
                    scatter_copy(b1).wait()

                @pl.when(k + 1 < NCH_PAIR)
                def _():
                    gather_copy(b1).start()

                @pl.when(k + 2 < NCH_PAIR)
                def _():
                    issue_idx(k + 2, b2)

                @pl.when(k < NCH_PAIR)
                def _():
                    gather_copy(b).wait()
                    scale(b)
                    scatter_copy(b).start(add=True)

        plsc.subcore_barrier()
        pltpu.sync_copy(
            acc_sh.at[pl.ds(s * ROWS_PER_TILE, ROWS_PER_TILE)], part_hbm.at[c, s]
        )

    return agg2_kernel(z_pair, row, col, edge_attr)


_BM = 2000


def _tc_dis(p0, p1, x, edge_attr):
    def body(attr_ref, p0_ref, p1_ref, x_ref, g_ref, dis2_ref, z0_ref, inv_ref):
        @pl.when(pl.program_id(0) == 0)
        def _():
            inv_ref[0] = 1.0 / jnp.max(attr_ref[...])

        inv_m = inv_ref[0]
        deg = (p0_ref[:, 0:1] + p1_ref[:, 0:1]) * inv_m + 2.0
        dis = lax.rsqrt(deg)
        g = dis * jnp.sqrt(inv_m)
        g_ref[...] = g
        dis2_ref[...] = 2.0 * dis * dis
        z0_ref[...] = g * x_ref[...]

    grid = (N_NODES // _BM,)
    return pl.pallas_call(
        body,
        grid=grid,
        in_specs=[
            pl.BlockSpec((2500, 128), lambda i: (0, 0)),
            pl.BlockSpec((_BM, 128), lambda i: (i, 0)),
            pl.BlockSpec((_BM, 128), lambda i: (i, 0)),
            pl.BlockSpec((_BM, 128), lambda i: (i, 0)),
        ],
        out_specs=[
            pl.BlockSpec((_BM, 1), lambda i: (i, 0)),
            pl.BlockSpec((_BM, 1), lambda i: (i, 0)),
            pl.BlockSpec((_BM, 128), lambda i: (i, 0)),
        ],
        out_shape=[
            jax.ShapeDtypeStruct((N_NODES, 1), jnp.float32),
            jax.ShapeDtypeStruct((N_NODES, 1), jnp.float32),
            jax.ShapeDtypeStruct((N_NODES, 128), jnp.float32),
        ],
        scratch_shapes=[pltpu.SMEM((1,), jnp.float32)],
    )(edge_attr.reshape(2500, 128), p0, p1, x)


def _tc_layer0(a0, a1, x, g, dis2, W0, b0, W1):
    def body(a0_ref, a1_ref, x_ref, g_ref, dis2_ref, W0_ref, b0_ref, W1_ref,
             v1_ref, z1_ref):
        u0 = g_ref[...] * (a0_ref[...] + a1_ref[...]) + dis2_ref[...] * x_ref[...]
        h1 = jnp.maximum(
            jnp.dot(u0, W0_ref[...], preferred_element_type=jnp.float32)
            + b0_ref[...],
            0.0,
        )
        v1 = jnp.dot(h1, W1_ref[...], preferred_element_type=jnp.float32)
        v1_ref[...] = v1
        z1 = g_ref[...] * v1
        z1_ref[0] = z1[:, :128]
        z1_ref[1] = z1[:, 128:]

    grid = (N_NODES // _BM,)
    return pl.pallas_call(
        body,
        grid=grid,
        in_specs=[
            pl.BlockSpec((_BM, 128), lambda i: (i, 0)),
            pl.BlockSpec((_BM, 128), lambda i: (i, 0)),
            pl.BlockSpec((_BM, 128), lambda i: (i, 0)),
            pl.BlockSpec((_BM, 1), lambda i: (i, 0)),
            pl.BlockSpec((_BM, 1), lambda i: (i, 0)),
            pl.BlockSpec((128, 256), lambda i: (0, 0)),
            pl.BlockSpec((1, 256), lambda i: (0, 0)),
            pl.BlockSpec((256, 256), lambda i: (0, 0)),
        ],
        out_specs=[
            pl.BlockSpec((_BM, 256), lambda i: (i, 0)),
            pl.BlockSpec((2, _BM, 128), lambda i: (0, i, 0)),
        ],
        out_shape=[
            jax.ShapeDtypeStruct((N_NODES, 256), jnp.float32),
            jax.ShapeDtypeStruct((2, N_NODES, 128), jnp.float32),
        ],
    )(a0, a1, x, g, dis2, W0, b0, W1)


def _tc_layer1(aa, ab, v1, g, dis2, b1, W2):
    def body(aa_ref, ab_ref, v1_ref, g_ref, dis2_ref, b1_ref, W2_ref,
             v2_ref, z2_ref):
        g_v = g_ref[...]
        d2 = dis2_ref[...]
        ua = g_v * aa_ref[...] + d2 * v1_ref[:, :128]
        ub = g_v * ab_ref[...] + d2 * v1_ref[:, 128:]
        h2 = jnp.maximum(
            jnp.concatenate([ua, ub], axis=1) + b1_ref[...], 0.0
        )
        v2 = jnp.dot(h2, W2_ref[...], preferred_element_type=jnp.float32)
        v2_ref[...] = v2
        z2_ref[...] = g_v * v2

    grid = (N_NODES // _BM,)
    return pl.pallas_call(
        body,
        grid=grid,
        in_specs=[
            pl.BlockSpec((_BM, 128), lambda i: (i, 0)),
            pl.BlockSpec((_BM, 128), lambda i: (i, 0)),
            pl.BlockSpec((_BM, 256), lambda i: (i, 0)),
            pl.BlockSpec((_BM, 1), lambda i: (i, 0)),
            pl.BlockSpec((_BM, 1), lambda i: (i, 0)),
            pl.BlockSpec((1, 256), lambda i: (0, 0)),
            pl.BlockSpec((256, 128), lambda i: (0, 0)),
        ],
        out_specs=[
            pl.BlockSpec((_BM, 128), lambda i: (i, 0)),
            pl.BlockSpec((_BM, 128), lambda i: (i, 0)),
        ],
        out_shape=[
            jax.ShapeDtypeStruct((N_NODES, 128), jnp.float32),
            jax.ShapeDtypeStruct((N_NODES, 128), jnp.float32),
        ],
    )(aa, ab, v1, g, dis2, b1, W2)


def _tc_layer2(a0, a1, v2, g, dis2, b2):
    def body(a0_ref, a1_ref, v2_ref, g_ref, dis2_ref, b2_ref, o_ref):
        o_ref[...] = (
            g_ref[...] * (a0_ref[...] + a1_ref[...])
            + dis2_ref[...] * v2_ref[...]
            + b2_ref[...]
        )

    grid = (N_NODES // _BM,)
    return pl.pallas_call(
        body,
        grid=grid,
        in_specs=[
            pl.BlockSpec((_BM, 128), lambda i: (i, 0)),
            pl.BlockSpec((_BM, 128), lambda i: (i, 0)),
            pl.BlockSpec((_BM, 128), lambda i: (i, 0)),
            pl.BlockSpec((_BM, 1), lambda i: (i, 0)),
            pl.BlockSpec((_BM, 1), lambda i: (i, 0)),
            pl.BlockSpec((1, 128), lambda i: (0, 0)),
        ],
        out_specs=pl.BlockSpec((_BM, 128), lambda i: (i, 0)),
        out_shape=jax.ShapeDtypeStruct((N_NODES, 128), jnp.float32),
    )(a0, a1, v2, g, dis2, b2)


def kernel(x, edge_index, edge_attr, W0, b0, W1, b1, W2, b2):
    row = edge_index[0].astype(jnp.int32)
    col = edge_index[1].astype(jnp.int32)

    degp = _sc_deg(edge_attr, col)
    degp = degp.reshape(NC, N_NODES, 128)
    g, dis2, z0 = _tc_dis(degp[0], degp[1], x, edge_attr)

    acc0 = _sc_aggregate(z0, row, col, edge_attr).reshape(NC, N_NODES, 128)
    v1, z1 = _tc_layer0(acc0[0], acc0[1], x, g, dis2, W0,
                        b0.reshape(1, 256), W1)

    acc1 = _sc_aggregate_pair(z1, row, col, edge_attr).reshape(NC, N_NODES, 128)
    v2, z2 = _tc_layer1(acc1[0], acc1[1], v1, g, dis2,
                        b1.reshape(1, 256), W2)

    acc2 = _sc_aggregate(z2, row, col, edge_attr).reshape(NC, N_NODES, 128)
    return _tc_layer2(acc2[0], acc2[1], v2, g, dis2, b2.reshape(1, 128))
